# Initial kernel scaffold; baseline (speedup 1.0000x reference)
import jax
import jax.numpy as jnp
from jax.experimental import pallas as pl


def kernel(x, edge_index, W1_self, W1_neigh, b1, W2_self, W2_neigh, b2):
    raise NotImplementedError("write your pallas kernel here")


# speedup vs baseline: 9.1829x; 9.1829x over previous
import functools

import jax
import jax.numpy as jnp
from jax import lax
from jax.experimental import pallas as pl
from jax.experimental.pallas import tpu as pltpu
from jax.experimental.pallas import tpu_sc as plsc

N = 10000
E = 320000
DIN = 128
DH = 32
NCLS = 40

NC = 2
NS = 16
LANES = 16
NW = NC * NS
NPAD = 10240
EPW = 10240
EPAD = NW * EPW
CH = 2048
NCHUNK = EPW // CH
RPT = NPAD // NS


def _sc_mesh():
    return plsc.VectorSubcoreMesh(
        core_axis_name="c", subcore_axis_name="s",
        num_cores=NC, num_subcores=NS)


def _zero_rows(rows_v, nrows, width):
    def body(i, _):
        for j in range(width // LANES):
            rows_v[i, pl.ds(j * LANES, LANES)] = jnp.zeros((LANES,), jnp.float32)
        return 0
    lax.fori_loop(0, nrows, body, 0)


def _fill_1d(vec_v, n, value):
    def body(i, _):
        vec_v[pl.ds(pl.multiple_of(i * LANES, LANES), LANES)] = jnp.full(
            (LANES,), value, jnp.float32)
        return 0
    lax.fori_loop(0, n // LANES, body, 0)


def _agg_deg_body(y_hbm, src_hbm, dst_hbm, agg_out, deg_out,
                  src_v, dst_v, rows_v, ones_v, zrow_v, acc_sh, deg_sh, sem):
    c = lax.axis_index("c")
    s = lax.axis_index("s")
    wid = c * NS + s

    _zero_rows(rows_v, RPT, DH)
    _fill_1d(ones_v, CH, 1.0)
    _fill_1d(zrow_v, RPT, 0.0)
    strip = pl.ds(pl.multiple_of(s * RPT, RPT), RPT)
    pltpu.sync_copy(rows_v.at[pl.ds(0, RPT)], acc_sh.at[strip])
    pltpu.sync_copy(zrow_v, deg_sh.at[strip])
    plsc.subcore_barrier()

    for k in range(NCHUNK):
        base = wid * EPW + k * CH
        pltpu.sync_copy(src_hbm.at[pl.ds(base, CH)], src_v)
        pltpu.sync_copy(dst_hbm.at[pl.ds(base, CH)], dst_v)
        pltpu.async_copy(y_hbm.at[src_v], rows_v, sem).wait()
        pltpu.sync_copy(rows_v, acc_sh.at[dst_v], add=True)
        pltpu.sync_copy(ones_v, deg_sh.at[dst_v], add=True)

    plsc.subcore_barrier()
    pltpu.sync_copy(acc_sh.at[strip], agg_out.at[c, strip])
    pltpu.sync_copy(deg_sh.at[strip], deg_out.at[c, strip])


def _agg_body(y_hbm, src_hbm, dst_hbm, agg_out,
              src_v, dst_v, rows_v, acc_sh, sem):
    c = lax.axis_index("c")
    s = lax.axis_index("s")
    wid = c * NS + s

    _zero_rows(rows_v, RPT, DH)
    strip = pl.ds(pl.multiple_of(s * RPT, RPT), RPT)
    pltpu.sync_copy(rows_v.at[pl.ds(0, RPT)], acc_sh.at[strip])
    plsc.subcore_barrier()

    for k in range(NCHUNK):
        base = wid * EPW + k * CH
        pltpu.sync_copy(src_hbm.at[pl.ds(base, CH)], src_v)
        pltpu.sync_copy(dst_hbm.at[pl.ds(base, CH)], dst_v)
        pltpu.async_copy(y_hbm.at[src_v], rows_v, sem).wait()
        pltpu.sync_copy(rows_v, acc_sh.at[dst_v], add=True)

    plsc.subcore_barrier()
    pltpu.sync_copy(acc_sh.at[strip], agg_out.at[c, strip])


def _agg_deg(y, src_p, dst_p):
    k = pl.kernel(
        _agg_deg_body,
        out_type=(jax.ShapeDtypeStruct((NC, NPAD, DH), jnp.float32),
                  jax.ShapeDtypeStruct((NC, NPAD), jnp.float32)),
        mesh=_sc_mesh(),
        compiler_params=pltpu.CompilerParams(use_tc_tiling_on_sc=False),
        scratch_types=[
            pltpu.VMEM((CH,), jnp.int32),
            pltpu.VMEM((CH,), jnp.int32),
            pltpu.VMEM((CH, DH), jnp.float32),
            pltpu.VMEM((CH,), jnp.float32),
            pltpu.VMEM((RPT,), jnp.float32),
            pltpu.VMEM_SHARED((NPAD, DH), jnp.float32),
            pltpu.VMEM_SHARED((NPAD,), jnp.float32),
            pltpu.SemaphoreType.DMA,
        ])
    return k(y, src_p, dst_p)


def _agg(y, src_p, dst_p):
    k = pl.kernel(
        _agg_body,
        out_type=jax.ShapeDtypeStruct((NC, NPAD, DH), jnp.float32),
        mesh=_sc_mesh(),
        compiler_params=pltpu.CompilerParams(use_tc_tiling_on_sc=False),
        scratch_types=[
            pltpu.VMEM((CH,), jnp.int32),
            pltpu.VMEM((CH,), jnp.int32),
            pltpu.VMEM((CH, DH), jnp.float32),
            pltpu.VMEM_SHARED((NPAD, DH), jnp.float32),
            pltpu.SemaphoreType.DMA,
        ])
    return k(y, src_p, dst_p)


RB = 1280


def _stage_a_kernel(x_ref, ws_ref, wn_ref, s1_ref, y1_ref):
    xb = x_ref[...]
    s1_ref[...] = jnp.dot(xb, ws_ref[...], preferred_element_type=jnp.float32)
    y1_ref[...] = jnp.dot(xb, wn_ref[...], preferred_element_type=jnp.float32)


def _stage_a(x_p, w_self, w_neigh):
    return pl.pallas_call(
        _stage_a_kernel,
        grid=(NPAD // RB,),
        in_specs=[pl.BlockSpec((RB, DIN), lambda i: (i, 0)),
                  pl.BlockSpec((DIN, DH), lambda i: (0, 0)),
                  pl.BlockSpec((DIN, DH), lambda i: (0, 0))],
        out_specs=[pl.BlockSpec((RB, DH), lambda i: (i, 0)),
                   pl.BlockSpec((RB, DH), lambda i: (i, 0))],
        out_shape=[jax.ShapeDtypeStruct((NPAD, DH), jnp.float32)] * 2,
    )(x_p, w_self, w_neigh)


def _stage_c_kernel(s1_ref, a0_ref, a1_ref, d0_ref, d1_ref, b1_ref, w2s_ref,
                    h_ref, s2_ref, rdeg_ref):
    deg = d0_ref[...] + d1_ref[...]
    rdeg = 1.0 / jnp.maximum(deg, 1.0)
    h = jnp.maximum(
        s1_ref[...] + (a0_ref[...] + a1_ref[...]) * rdeg + b1_ref[...], 0.0)
    h_ref[...] = h
    s2_ref[...] = jnp.dot(h, w2s_ref[...], preferred_element_type=jnp.float32)
    rdeg_ref[...] = rdeg


def _stage_c(s1, a0, a1, d0, d1, b1, w2_self):
    return pl.pallas_call(
        _stage_c_kernel,
        grid=(NPAD // RB,),
        in_specs=[pl.BlockSpec((RB, DH), lambda i: (i, 0)),
                  pl.BlockSpec((RB, DH), lambda i: (i, 0)),
                  pl.BlockSpec((RB, DH), lambda i: (i, 0)),
                  pl.BlockSpec((RB, 1), lambda i: (i, 0)),
                  pl.BlockSpec((RB, 1), lambda i: (i, 0)),
                  pl.BlockSpec((1, DH), lambda i: (0, 0)),
                  pl.BlockSpec((DH, NCLS), lambda i: (0, 0))],
        out_specs=[pl.BlockSpec((RB, DH), lambda i: (i, 0)),
                   pl.BlockSpec((RB, NCLS), lambda i: (i, 0)),
                   pl.BlockSpec((RB, 1), lambda i: (i, 0))],
        out_shape=[jax.ShapeDtypeStruct((NPAD, DH), jnp.float32),
                   jax.ShapeDtypeStruct((NPAD, NCLS), jnp.float32),
                   jax.ShapeDtypeStruct((NPAD, 1), jnp.float32)],
    )(s1, a0, a1, d0, d1, b1, w2_self)


def _stage_e_kernel(s2_ref, a0_ref, a1_ref, rdeg_ref, w2n_ref, b2_ref, o_ref):
    mean = (a0_ref[...] + a1_ref[...]) * rdeg_ref[...]
    t = s2_ref[...] + jnp.dot(mean, w2n_ref[...],
                              preferred_element_type=jnp.float32) + b2_ref[...]
    m = jnp.max(t, axis=1, keepdims=True)
    lse = m + jnp.log(jnp.sum(jnp.exp(t - m), axis=1, keepdims=True))
    o_ref[...] = t - lse


def _stage_e(s2, a0, a1, rdeg, w2_neigh, b2):
    return pl.pallas_call(
        _stage_e_kernel,
        grid=(NPAD // RB,),
        in_specs=[pl.BlockSpec((RB, NCLS), lambda i: (i, 0)),
                  pl.BlockSpec((RB, DH), lambda i: (i, 0)),
                  pl.BlockSpec((RB, DH), lambda i: (i, 0)),
                  pl.BlockSpec((RB, 1), lambda i: (i, 0)),
                  pl.BlockSpec((DH, NCLS), lambda i: (0, 0)),
                  pl.BlockSpec((1, NCLS), lambda i: (0, 0))],
        out_specs=pl.BlockSpec((RB, NCLS), lambda i: (i, 0)),
        out_shape=jax.ShapeDtypeStruct((NPAD, NCLS), jnp.float32),
    )(s2, a0, a1, rdeg, w2_neigh, b2)


def kernel(x, edge_index, W1_self, W1_neigh, b1, W2_self, W2_neigh, b2):
    src = edge_index[0].astype(jnp.int32)
    dst = edge_index[1].astype(jnp.int32)
    padlen = EPAD - E
    pad_idx = jnp.full((padlen,), N, jnp.int32)
    src_p = jnp.concatenate([src, pad_idx])
    dst_p = jnp.concatenate([dst, pad_idx])
    x_p = jnp.pad(x, ((0, NPAD - N), (0, 0)))

    s1, y1 = _stage_a(x_p, W1_self, W1_neigh)
    agg1, degp = _agg_deg(y1, src_p, dst_p)
    h, s2, rdeg = _stage_c(s1, agg1[0], agg1[1],
                           degp[0][:, None], degp[1][:, None],
                           b1[None, :], W2_self)
    agg2 = _agg(h, src_p, dst_p)
    out = _stage_e(s2, agg2[0], agg2[1], rdeg, W2_neigh, b2[None, :])
    return out[:N]


# speedup vs baseline: 10.4842x; 1.1417x over previous
import functools

import jax
import jax.numpy as jnp
from jax import lax
from jax.experimental import pallas as pl
from jax.experimental.pallas import tpu as pltpu
from jax.experimental.pallas import tpu_sc as plsc

N = 10000
E = 320000
DIN = 128
DH = 32
NCLS = 40

NC = 2
NS = 16
LANES = 16
NW = NC * NS
NPAD = 10240
EPW = 10240
EPAD = NW * EPW
CH = 1280
NCHUNK = EPW // CH
RPT = NPAD // NS


def _sc_mesh():
    return plsc.VectorSubcoreMesh(
        core_axis_name="c", subcore_axis_name="s",
        num_cores=NC, num_subcores=NS)


def _zero_rows(rows_v, nrows, width):
    z = jnp.zeros((LANES,), jnp.float32)
    def body(i, _):
        r4 = pl.multiple_of(i * 4, 4)
        for r in range(4):
            for j in range(width // LANES):
                rows_v[r4 + r, pl.ds(j * LANES, LANES)] = z
        return 0
    lax.fori_loop(0, nrows // 4, body, 0)


def _fill_1d(vec_v, n, value):
    v = jnp.full((LANES,), value, jnp.float32)
    def body(i, _):
        vec_v[pl.ds(pl.multiple_of(i * LANES, LANES), LANES)] = v
        return 0
    lax.fori_loop(0, n // LANES, body, 0)


def _edge_pipeline(y_hbm, src2_hbm, dst2_hbm, acc_sh, deg_sh,
                   idx_v, rows0, rows1, ones_v, zrow_v,
                   isem, gsem0, gsem1, s):
    c = lax.axis_index("c")
    wid = c * NS + s

    cp_idx = pltpu.async_copy(
        src2_hbm.at[pl.ds(wid * NCHUNK, NCHUNK), :], idx_v.at[0], isem)
    cp_idx2 = pltpu.async_copy(
        dst2_hbm.at[pl.ds(wid * NCHUNK, NCHUNK), :], idx_v.at[1], isem)

    _zero_rows(rows0, RPT, DH)
    strip = pl.ds(pl.multiple_of(s * RPT, RPT), RPT)
    pltpu.sync_copy(rows0.at[pl.ds(0, RPT)], acc_sh.at[strip])
    if deg_sh is not None:
        _fill_1d(ones_v, CH, 1.0)
        _fill_1d(zrow_v, RPT, 0.0)
        pltpu.sync_copy(zrow_v, deg_sh.at[strip])
    plsc.subcore_barrier()
    cp_idx.wait()
    cp_idx2.wait()

    bufs = (rows0, rows1)
    gsems = (gsem0, gsem1)
    gathers = [None] * NCHUNK
    gathers[0] = pltpu.async_copy(y_hbm.at[idx_v.at[0].at[0]], bufs[0], gsem0)
    for k in range(NCHUNK):
        p = k % 2
        if k + 1 < NCHUNK:
            gathers[k + 1] = pltpu.async_copy(
                y_hbm.at[idx_v.at[0].at[k + 1]], bufs[1 - p], gsems[1 - p])
        gathers[k].wait()
        pltpu.sync_copy(bufs[p], acc_sh.at[idx_v.at[1].at[k]], add=True)
        if deg_sh is not None:
            pltpu.sync_copy(ones_v, deg_sh.at[idx_v.at[1].at[k]], add=True)
    plsc.subcore_barrier()
    return strip


def _agg_deg_body(y_hbm, src2_hbm, dst2_hbm, agg_out, deg_out,
                  idx_v, rows0, rows1, ones_v, zrow_v, acc_sh, deg_sh,
                  isem, gsem0, gsem1):
    c = lax.axis_index("c")
    s = lax.axis_index("s")
    strip = _edge_pipeline(y_hbm, src2_hbm, dst2_hbm, acc_sh, deg_sh,
                           idx_v, rows0, rows1, ones_v, zrow_v,
                           isem, gsem0, gsem1, s)
    pltpu.sync_copy(acc_sh.at[strip], agg_out.at[c, strip])
    pltpu.sync_copy(deg_sh.at[strip], deg_out.at[c, strip])


def _agg_body(y_hbm, src2_hbm, dst2_hbm, agg_out,
              idx_v, rows0, rows1, acc_sh, isem, gsem0, gsem1):
    c = lax.axis_index("c")
    s = lax.axis_index("s")
    strip = _edge_pipeline(y_hbm, src2_hbm, dst2_hbm, acc_sh, None,
                           idx_v, rows0, rows1, None, None,
                           isem, gsem0, gsem1, s)
    pltpu.sync_copy(acc_sh.at[strip], agg_out.at[c, strip])


def _agg_deg(y, src2, dst2):
    k = pl.kernel(
        _agg_deg_body,
        out_type=(jax.ShapeDtypeStruct((NC, NPAD, DH), jnp.float32),
                  jax.ShapeDtypeStruct((NC, NPAD), jnp.float32)),
        mesh=_sc_mesh(),
        compiler_params=pltpu.CompilerParams(use_tc_tiling_on_sc=False),
        scratch_types=[
            pltpu.VMEM((2, NCHUNK, CH), jnp.int32),
            pltpu.VMEM((CH, DH), jnp.float32),
            pltpu.VMEM((CH, DH), jnp.float32),
            pltpu.VMEM((CH,), jnp.float32),
            pltpu.VMEM((RPT,), jnp.float32),
            pltpu.VMEM_SHARED((NPAD, DH), jnp.float32),
            pltpu.VMEM_SHARED((NPAD,), jnp.float32),
            pltpu.SemaphoreType.DMA,
            pltpu.SemaphoreType.DMA,
            pltpu.SemaphoreType.DMA,
        ])
    return k(y, src2, dst2)


def _agg(y, src2, dst2):
    k = pl.kernel(
        _agg_body,
        out_type=jax.ShapeDtypeStruct((NC, NPAD, DH), jnp.float32),
        mesh=_sc_mesh(),
        compiler_params=pltpu.CompilerParams(use_tc_tiling_on_sc=False),
        scratch_types=[
            pltpu.VMEM((2, NCHUNK, CH), jnp.int32),
            pltpu.VMEM((CH, DH), jnp.float32),
            pltpu.VMEM((CH, DH), jnp.float32),
            pltpu.VMEM_SHARED((NPAD, DH), jnp.float32),
            pltpu.SemaphoreType.DMA,
            pltpu.SemaphoreType.DMA,
            pltpu.SemaphoreType.DMA,
        ])
    return k(y, src2, dst2)


RB = 1280


def _stage_a_kernel(x_ref, ws_ref, wn_ref, s1_ref, y1_ref):
    xb = x_ref[...]
    s1_ref[...] = jnp.dot(xb, ws_ref[...], preferred_element_type=jnp.float32)
    y1_ref[...] = jnp.dot(xb, wn_ref[...], preferred_element_type=jnp.float32)


def _stage_a(x_p, w_self, w_neigh):
    return pl.pallas_call(
        _stage_a_kernel,
        grid=(NPAD // RB,),
        in_specs=[pl.BlockSpec((RB, DIN), lambda i: (i, 0)),
                  pl.BlockSpec((DIN, DH), lambda i: (0, 0)),
                  pl.BlockSpec((DIN, DH), lambda i: (0, 0))],
        out_specs=[pl.BlockSpec((RB, DH), lambda i: (i, 0)),
                   pl.BlockSpec((RB, DH), lambda i: (i, 0))],
        out_shape=[jax.ShapeDtypeStruct((NPAD, DH), jnp.float32)] * 2,
    )(x_p, w_self, w_neigh)


def _stage_c_kernel(s1_ref, a0_ref, a1_ref, d0_ref, d1_ref, b1_ref, w2s_ref,
                    h_ref, s2_ref, rdeg_ref):
    deg = d0_ref[...] + d1_ref[...]
    rdeg = 1.0 / jnp.maximum(deg, 1.0)
    h = jnp.maximum(
        s1_ref[...] + (a0_ref[...] + a1_ref[...]) * rdeg + b1_ref[...], 0.0)
    h_ref[...] = h
    s2_ref[...] = jnp.dot(h, w2s_ref[...], preferred_element_type=jnp.float32)
    rdeg_ref[...] = rdeg


def _stage_c(s1, a0, a1, d0, d1, b1, w2_self):
    return pl.pallas_call(
        _stage_c_kernel,
        grid=(NPAD // RB,),
        in_specs=[pl.BlockSpec((RB, DH), lambda i: (i, 0)),
                  pl.BlockSpec((RB, DH), lambda i: (i, 0)),
                  pl.BlockSpec((RB, DH), lambda i: (i, 0)),
                  pl.BlockSpec((RB, 1), lambda i: (i, 0)),
                  pl.BlockSpec((RB, 1), lambda i: (i, 0)),
                  pl.BlockSpec((1, DH), lambda i: (0, 0)),
                  pl.BlockSpec((DH, NCLS), lambda i: (0, 0))],
        out_specs=[pl.BlockSpec((RB, DH), lambda i: (i, 0)),
                   pl.BlockSpec((RB, NCLS), lambda i: (i, 0)),
                   pl.BlockSpec((RB, 1), lambda i: (i, 0))],
        out_shape=[jax.ShapeDtypeStruct((NPAD, DH), jnp.float32),
                   jax.ShapeDtypeStruct((NPAD, NCLS), jnp.float32),
                   jax.ShapeDtypeStruct((NPAD, 1), jnp.float32)],
    )(s1, a0, a1, d0, d1, b1, w2_self)


def _stage_e_kernel(s2_ref, a0_ref, a1_ref, rdeg_ref, w2n_ref, b2_ref, o_ref):
    mean = (a0_ref[...] + a1_ref[...]) * rdeg_ref[...]
    t = s2_ref[...] + jnp.dot(mean, w2n_ref[...],
                              preferred_element_type=jnp.float32) + b2_ref[...]
    m = jnp.max(t, axis=1, keepdims=True)
    lse = m + jnp.log(jnp.sum(jnp.exp(t - m), axis=1, keepdims=True))
    o_ref[...] = t - lse


def _stage_e(s2, a0, a1, rdeg, w2_neigh, b2):
    return pl.pallas_call(
        _stage_e_kernel,
        grid=(NPAD // RB,),
        in_specs=[pl.BlockSpec((RB, NCLS), lambda i: (i, 0)),
                  pl.BlockSpec((RB, DH), lambda i: (i, 0)),
                  pl.BlockSpec((RB, DH), lambda i: (i, 0)),
                  pl.BlockSpec((RB, 1), lambda i: (i, 0)),
                  pl.BlockSpec((DH, NCLS), lambda i: (0, 0)),
                  pl.BlockSpec((1, NCLS), lambda i: (0, 0))],
        out_specs=pl.BlockSpec((RB, NCLS), lambda i: (i, 0)),
        out_shape=jax.ShapeDtypeStruct((NPAD, NCLS), jnp.float32),
    )(s2, a0, a1, rdeg, w2_neigh, b2)


def kernel(x, edge_index, W1_self, W1_neigh, b1, W2_self, W2_neigh, b2):
    src = edge_index[0].astype(jnp.int32)
    dst = edge_index[1].astype(jnp.int32)
    padlen = EPAD - E
    pad_idx = jnp.full((padlen,), N, jnp.int32)
    src_p = jnp.concatenate([src, pad_idx]).reshape(NW * NCHUNK, CH)
    dst_p = jnp.concatenate([dst, pad_idx]).reshape(NW * NCHUNK, CH)
    x_p = jnp.pad(x, ((0, NPAD - N), (0, 0)))

    s1, y1 = _stage_a(x_p, W1_self, W1_neigh)
    agg1, degp = _agg_deg(y1, src_p, dst_p)
    h, s2, rdeg = _stage_c(s1, agg1[0], agg1[1],
                           degp[0][:, None], degp[1][:, None],
                           b1[None, :], W2_self)
    agg2 = _agg(h, src_p, dst_p)
    out = _stage_e(s2, agg2[0], agg2[1], rdeg, W2_neigh, b2[None, :])
    return out[:N]


# speedup vs baseline: 19.8580x; 1.8941x over previous
import functools

import jax
import jax.numpy as jnp
from jax import lax
from jax.experimental import pallas as pl
from jax.experimental.pallas import tpu as pltpu
from jax.experimental.pallas import tpu_sc as plsc

N = 10000
E = 320000
DIN = 128
DH = 32
NCLS = 40

NC = 2
NS = 16
LANES = 16
NW = NC * NS
NPAD = 10240
EPW = 10240
EPAD = NW * EPW
CH = 1280
NCHUNK = EPW // CH
RPT = NPAD // NS


def _sc_mesh():
    return plsc.VectorSubcoreMesh(
        core_axis_name="c", subcore_axis_name="s",
        num_cores=NC, num_subcores=NS)


def _zero_rows(rows_v, nrows, width):
    z = jnp.zeros((LANES,), jnp.float32)
    def body(i, _):
        r4 = pl.multiple_of(i * 4, 4)
        for r in range(4):
            for j in range(width // LANES):
                rows_v[r4 + r, pl.ds(j * LANES, LANES)] = z
        return 0
    lax.fori_loop(0, nrows // 4, body, 0)


def _fill_1d(vec_v, n, value):
    v = jnp.full((LANES,), value, jnp.float32)
    def body(i, _):
        vec_v[pl.ds(pl.multiple_of(i * LANES, LANES), LANES)] = v
        return 0
    lax.fori_loop(0, n // LANES, body, 0)


def _edge_pipeline(y_hbm, src2_hbm, dst2_hbm, acc_sh, deg_sh,
                   idx_v, rows0, rows1, ones_v, zrow_v,
                   isem, gsem0, gsem1, s):
    c = lax.axis_index("c")
    wid = c * NS + s

    cp_idx = pltpu.async_copy(
        src2_hbm.at[pl.ds(wid * NCHUNK, NCHUNK), :], idx_v.at[0], isem)
    cp_idx2 = pltpu.async_copy(
        dst2_hbm.at[pl.ds(wid * NCHUNK, NCHUNK), :], idx_v.at[1], isem)

    _zero_rows(rows0, RPT, DH)
    strip = pl.ds(pl.multiple_of(s * RPT, RPT), RPT)
    pltpu.sync_copy(rows0.at[pl.ds(0, RPT)], acc_sh.at[strip])
    if deg_sh is not None:
        _fill_1d(ones_v, CH, 1.0)
        _fill_1d(zrow_v, RPT, 0.0)
        pltpu.sync_copy(zrow_v, deg_sh.at[strip])
    plsc.subcore_barrier()
    cp_idx.wait()
    cp_idx2.wait()

    bufs = (rows0, rows1)
    gsems = (gsem0, gsem1)
    gathers = [None] * NCHUNK
    gathers[0] = pltpu.async_copy(y_hbm.at[idx_v.at[0].at[0]], bufs[0], gsem0)
    for k in range(NCHUNK):
        p = k % 2
        if k + 1 < NCHUNK:
            gathers[k + 1] = pltpu.async_copy(
                y_hbm.at[idx_v.at[0].at[k + 1]], bufs[1 - p], gsems[1 - p])
        gathers[k].wait()
        pltpu.sync_copy(bufs[p], acc_sh.at[idx_v.at[1].at[k]], add=True)
        if deg_sh is not None:
            pltpu.sync_copy(ones_v, deg_sh.at[idx_v.at[1].at[k]], add=True)
    plsc.subcore_barrier()
    return strip


def _agg_deg_body(y_hbm, src2_hbm, dst2_hbm, agg_out, deg_out,
                  idx_v, rows0, rows1, ones_v, zrow_v, acc_sh, deg_sh,
                  isem, gsem0, gsem1):
    c = lax.axis_index("c")
    s = lax.axis_index("s")
    strip = _edge_pipeline(y_hbm, src2_hbm, dst2_hbm, acc_sh, deg_sh,
                           idx_v, rows0, rows1, ones_v, zrow_v,
                           isem, gsem0, gsem1, s)
    pltpu.sync_copy(acc_sh.at[strip], agg_out.at[c, strip])
    pltpu.sync_copy(deg_sh.at[strip], deg_out.at[c, strip])


def _agg_body(y_hbm, src2_hbm, dst2_hbm, agg_out,
              idx_v, rows0, rows1, acc_sh, isem, gsem0, gsem1):
    c = lax.axis_index("c")
    s = lax.axis_index("s")
    strip = _edge_pipeline(y_hbm, src2_hbm, dst2_hbm, acc_sh, None,
                           idx_v, rows0, rows1, None, None,
                           isem, gsem0, gsem1, s)
    pltpu.sync_copy(acc_sh.at[strip], agg_out.at[c, strip])


def _agg_deg(y, src2, dst2):
    k = pl.kernel(
        _agg_deg_body,
        out_type=(jax.ShapeDtypeStruct((NC, NPAD, DH), jnp.float32),
                  jax.ShapeDtypeStruct((NC, NPAD), jnp.float32)),
        mesh=_sc_mesh(),
        compiler_params=pltpu.CompilerParams(use_tc_tiling_on_sc=False),
        scratch_types=[
            pltpu.VMEM((2, NCHUNK, CH), jnp.int32),
            pltpu.VMEM((CH, DH), jnp.float32),
            pltpu.VMEM((CH, DH), jnp.float32),
            pltpu.VMEM((CH,), jnp.float32),
            pltpu.VMEM((RPT,), jnp.float32),
            pltpu.VMEM_SHARED((NPAD, DH), jnp.float32),
            pltpu.VMEM_SHARED((NPAD,), jnp.float32),
            pltpu.SemaphoreType.DMA,
            pltpu.SemaphoreType.DMA,
            pltpu.SemaphoreType.DMA,
        ])
    return k(y, src2, dst2)


def _agg(y, src2, dst2):
    k = pl.kernel(
        _agg_body,
        out_type=jax.ShapeDtypeStruct((NC, NPAD, DH), jnp.float32),
        mesh=_sc_mesh(),
        compiler_params=pltpu.CompilerParams(use_tc_tiling_on_sc=False),
        scratch_types=[
            pltpu.VMEM((2, NCHUNK, CH), jnp.int32),
            pltpu.VMEM((CH, DH), jnp.float32),
            pltpu.VMEM((CH, DH), jnp.float32),
            pltpu.VMEM_SHARED((NPAD, DH), jnp.float32),
            pltpu.SemaphoreType.DMA,
            pltpu.SemaphoreType.DMA,
            pltpu.SemaphoreType.DMA,
        ])
    return k(y, src2, dst2)


RB = 1280


def _stage_a_kernel(x_ref, ws_ref, wn_ref, s1_ref, y1_ref):
    xb = x_ref[...]
    s1_ref[...] = jnp.dot(xb, ws_ref[...], preferred_element_type=jnp.float32)
    y1_ref[...] = jnp.dot(xb, wn_ref[...], preferred_element_type=jnp.float32)


def _stage_a(x_p, w_self, w_neigh):
    return pl.pallas_call(
        _stage_a_kernel,
        grid=(NPAD // RB,),
        in_specs=[pl.BlockSpec((RB, DIN), lambda i: (i, 0)),
                  pl.BlockSpec((DIN, DH), lambda i: (0, 0)),
                  pl.BlockSpec((DIN, DH), lambda i: (0, 0))],
        out_specs=[pl.BlockSpec((RB, DH), lambda i: (i, 0)),
                   pl.BlockSpec((RB, DH), lambda i: (i, 0))],
        out_shape=[jax.ShapeDtypeStruct((NPAD, DH), jnp.float32)] * 2,
    )(x_p, w_self, w_neigh)


def _stage_c_kernel(s1_ref, a0_ref, a1_ref, d0_ref, d1_ref, b1_ref, w2s_ref,
                    h_ref, s2_ref, rdeg_ref):
    deg = d0_ref[...] + d1_ref[...]
    rdeg = 1.0 / jnp.maximum(deg, 1.0)
    h = jnp.maximum(
        s1_ref[...] + (a0_ref[...] + a1_ref[...]) * rdeg + b1_ref[...], 0.0)
    h_ref[...] = h
    s2_ref[...] = jnp.dot(h, w2s_ref[...], preferred_element_type=jnp.float32)
    rdeg_ref[...] = rdeg


def _stage_c(s1, a0, a1, d0, d1, b1, w2_self):
    return pl.pallas_call(
        _stage_c_kernel,
        grid=(NPAD // RB,),
        in_specs=[pl.BlockSpec((RB, DH), lambda i: (i, 0)),
                  pl.BlockSpec((RB, DH), lambda i: (i, 0)),
                  pl.BlockSpec((RB, DH), lambda i: (i, 0)),
                  pl.BlockSpec((RB, 1), lambda i: (i, 0)),
                  pl.BlockSpec((RB, 1), lambda i: (i, 0)),
                  pl.BlockSpec((1, DH), lambda i: (0, 0)),
                  pl.BlockSpec((DH, NCLS), lambda i: (0, 0))],
        out_specs=[pl.BlockSpec((RB, DH), lambda i: (i, 0)),
                   pl.BlockSpec((RB, NCLS), lambda i: (i, 0)),
                   pl.BlockSpec((RB, 1), lambda i: (i, 0))],
        out_shape=[jax.ShapeDtypeStruct((NPAD, DH), jnp.float32),
                   jax.ShapeDtypeStruct((NPAD, NCLS), jnp.float32),
                   jax.ShapeDtypeStruct((NPAD, 1), jnp.float32)],
    )(s1, a0, a1, d0, d1, b1, w2_self)


def _stage_e_kernel(s2_ref, a0_ref, a1_ref, rdeg_ref, w2n_ref, b2_ref, o_ref):
    mean = (a0_ref[...] + a1_ref[...]) * rdeg_ref[...]
    t = s2_ref[...] + jnp.dot(mean, w2n_ref[...],
                              preferred_element_type=jnp.float32) + b2_ref[...]
    m = jnp.max(t, axis=1, keepdims=True)
    lse = m + jnp.log(jnp.sum(jnp.exp(t - m), axis=1, keepdims=True))
    o_ref[...] = t - lse


def _stage_e(s2, a0, a1, rdeg, w2_neigh, b2):
    return pl.pallas_call(
        _stage_e_kernel,
        grid=(NPAD // RB,),
        in_specs=[pl.BlockSpec((RB, NCLS), lambda i: (i, 0)),
                  pl.BlockSpec((RB, DH), lambda i: (i, 0)),
                  pl.BlockSpec((RB, DH), lambda i: (i, 0)),
                  pl.BlockSpec((RB, 1), lambda i: (i, 0)),
                  pl.BlockSpec((DH, NCLS), lambda i: (0, 0)),
                  pl.BlockSpec((1, NCLS), lambda i: (0, 0))],
        out_specs=pl.BlockSpec((RB, NCLS), lambda i: (i, 0)),
        out_shape=jax.ShapeDtypeStruct((NPAD, NCLS), jnp.float32),
    )(s2, a0, a1, rdeg, w2_neigh, b2)


def kernel(x, edge_index, W1_self, W1_neigh, b1, W2_self, W2_neigh, b2):
    src = edge_index[0].astype(jnp.int32)
    dst = edge_index[1].astype(jnp.int32)
    padlen = EPAD - E
    pad_idx = N + jnp.arange(padlen, dtype=jnp.int32) % (NPAD - N)
    src_p = jnp.concatenate([src, pad_idx]).reshape(NW * NCHUNK, CH)
    dst_p = jnp.concatenate([dst, pad_idx]).reshape(NW * NCHUNK, CH)
    x_p = jnp.pad(x, ((0, NPAD - N), (0, 0)))

    s1, y1 = _stage_a(x_p, W1_self, W1_neigh)
    agg1, degp = _agg_deg(y1, src_p, dst_p)
    h, s2, rdeg = _stage_c(s1, agg1[0], agg1[1],
                           degp[0][:, None], degp[1][:, None],
                           b1[None, :], W2_self)
    agg2 = _agg(h, src_p, dst_p)
    out = _stage_e(s2, agg2[0], agg2[1], rdeg, W2_neigh, b2[None, :])
    return out[:N]


# speedup vs baseline: 22.1019x; 1.1130x over previous
import jax
import jax.numpy as jnp
from jax import lax
from jax.experimental import pallas as pl
from jax.experimental.pallas import tpu as pltpu
from jax.experimental.pallas import tpu_sc as plsc

N = 10000
E = 320000
DIN = 128
DH = 32
NCLS = 40

NC = 2
NS = 16
LANES = 16
NW = NC * NS
NPAD = 10240
EPW = 10240
EPAD = NW * EPW
CH = 1280
NCHUNK = EPW // CH
RPT = NPAD // NS


def _sc_mesh():
    return plsc.VectorSubcoreMesh(
        core_axis_name="c", subcore_axis_name="s",
        num_cores=NC, num_subcores=NS)


def _zero_rows(rows_v, nrows, width):
    z = jnp.zeros((LANES,), jnp.float32)
    def body(i, _):
        r4 = pl.multiple_of(i * 4, 4)
        for r in range(4):
            for j in range(width // LANES):
                rows_v[r4 + r, pl.ds(j * LANES, LANES)] = z
        return 0
    lax.fori_loop(0, nrows // 4, body, 0)


def _fill_1d(vec_v, n, value):
    v = jnp.full((LANES,), value, jnp.float32)
    def body(i, _):
        vec_v[pl.ds(pl.multiple_of(i * LANES, LANES), LANES)] = v
        return 0
    lax.fori_loop(0, n // LANES, body, 0)


def _edge_pipeline(y_hbm, src_hbm, dst_hbm, acc_sh, deg_sh,
                   idx_v, rows0, rows1, ones_v, zrow_v,
                   isem, gsem0, gsem1, s):
    c = lax.axis_index("c")
    wid = c * NS + s

    idx_copies = []
    for k in range(NCHUNK):
        base = wid * EPW + k * CH
        idx_copies.append(pltpu.async_copy(
            src_hbm.at[pl.ds(base, CH)], idx_v.at[0].at[k], isem))
        idx_copies.append(pltpu.async_copy(
            dst_hbm.at[pl.ds(base, CH)], idx_v.at[1].at[k], isem))

    _zero_rows(rows0, RPT, DH)
    strip = pl.ds(pl.multiple_of(s * RPT, RPT), RPT)
    pltpu.sync_copy(rows0.at[pl.ds(0, RPT)], acc_sh.at[strip])
    if deg_sh is not None:
        _fill_1d(ones_v, CH, 1.0)
        _fill_1d(zrow_v, RPT, 0.0)
        pltpu.sync_copy(zrow_v, deg_sh.at[strip])
    plsc.subcore_barrier()
    for cp in idx_copies:
        cp.wait()

    bufs = (rows0, rows1)
    gsems = (gsem0, gsem1)
    gathers = [None] * NCHUNK
    gathers[0] = pltpu.async_copy(y_hbm.at[idx_v.at[0].at[0]], bufs[0], gsem0)
    for k in range(NCHUNK):
        p = k % 2
        if k + 1 < NCHUNK:
            gathers[k + 1] = pltpu.async_copy(
                y_hbm.at[idx_v.at[0].at[k + 1]], bufs[1 - p], gsems[1 - p])
        gathers[k].wait()
        pltpu.sync_copy(bufs[p], acc_sh.at[idx_v.at[1].at[k]], add=True)
        if deg_sh is not None:
            pltpu.sync_copy(ones_v, deg_sh.at[idx_v.at[1].at[k]], add=True)
    plsc.subcore_barrier()
    return strip


def _agg_deg_body(y_hbm, src_hbm, dst_hbm, agg_out, deg_out,
                  idx_v, rows0, rows1, ones_v, zrow_v, acc_sh, deg_sh,
                  isem, gsem0, gsem1):
    c = lax.axis_index("c")
    s = lax.axis_index("s")
    strip = _edge_pipeline(y_hbm, src_hbm, dst_hbm, acc_sh, deg_sh,
                           idx_v, rows0, rows1, ones_v, zrow_v,
                           isem, gsem0, gsem1, s)
    pltpu.sync_copy(acc_sh.at[strip], agg_out.at[c, strip])
    pltpu.sync_copy(deg_sh.at[strip], deg_out.at[c, strip])


def _agg_body(y_hbm, src_hbm, dst_hbm, agg_out,
              idx_v, rows0, rows1, acc_sh, isem, gsem0, gsem1):
    c = lax.axis_index("c")
    s = lax.axis_index("s")
    strip = _edge_pipeline(y_hbm, src_hbm, dst_hbm, acc_sh, None,
                           idx_v, rows0, rows1, None, None,
                           isem, gsem0, gsem1, s)
    pltpu.sync_copy(acc_sh.at[strip], agg_out.at[c, strip])


def _agg_deg(y, src_p, dst_p):
    k = pl.kernel(
        _agg_deg_body,
        out_type=(jax.ShapeDtypeStruct((NC, NPAD, DH), jnp.float32),
                  jax.ShapeDtypeStruct((NC, NPAD), jnp.float32)),
        mesh=_sc_mesh(),
        compiler_params=pltpu.CompilerParams(use_tc_tiling_on_sc=False),
        scratch_types=[
            pltpu.VMEM((2, NCHUNK, CH), jnp.int32),
            pltpu.VMEM((CH, DH), jnp.float32),
            pltpu.VMEM((CH, DH), jnp.float32),
            pltpu.VMEM((CH,), jnp.float32),
            pltpu.VMEM((RPT,), jnp.float32),
            pltpu.VMEM_SHARED((NPAD, DH), jnp.float32),
            pltpu.VMEM_SHARED((NPAD,), jnp.float32),
            pltpu.SemaphoreType.DMA,
            pltpu.SemaphoreType.DMA,
            pltpu.SemaphoreType.DMA,
        ])
    return k(y, src_p, dst_p)


def _agg(y, src_p, dst_p):
    k = pl.kernel(
        _agg_body,
        out_type=jax.ShapeDtypeStruct((NC, NPAD, DH), jnp.float32),
        mesh=_sc_mesh(),
        compiler_params=pltpu.CompilerParams(use_tc_tiling_on_sc=False),
        scratch_types=[
            pltpu.VMEM((2, NCHUNK, CH), jnp.int32),
            pltpu.VMEM((CH, DH), jnp.float32),
            pltpu.VMEM((CH, DH), jnp.float32),
            pltpu.VMEM_SHARED((NPAD, DH), jnp.float32),
            pltpu.SemaphoreType.DMA,
            pltpu.SemaphoreType.DMA,
            pltpu.SemaphoreType.DMA,
        ])
    return k(y, src_p, dst_p)


RB = 2000
GRID = N // RB


def _stage_a_kernel(x_ref, ws_ref, wn_ref, s1_ref, y1_ref):
    xb = x_ref[...]
    s1_ref[...] = jnp.dot(xb, ws_ref[...], preferred_element_type=jnp.float32)
    y1_ref[...] = jnp.dot(xb, wn_ref[...], preferred_element_type=jnp.float32)


def _stage_a(x, w_self, w_neigh):
    return pl.pallas_call(
        _stage_a_kernel,
        grid=(GRID,),
        in_specs=[pl.BlockSpec((RB, DIN), lambda i: (i, 0)),
                  pl.BlockSpec((DIN, DH), lambda i: (0, 0)),
                  pl.BlockSpec((DIN, DH), lambda i: (0, 0))],
        out_specs=[pl.BlockSpec((RB, DH), lambda i: (i, 0)),
                   pl.BlockSpec((RB, DH), lambda i: (i, 0))],
        out_shape=[jax.ShapeDtypeStruct((N, DH), jnp.float32)] * 2,
    )(x, w_self, w_neigh)


def _stage_c_kernel(s1_ref, agg_ref, d0_ref, d1_ref, b1_ref, w2s_ref,
                    h_ref, s2_ref, rdeg_ref):
    deg = d0_ref[...] + d1_ref[...]
    rdeg = 1.0 / jnp.maximum(deg, 1.0)
    h = jnp.maximum(
        s1_ref[...] + (agg_ref[0] + agg_ref[1]) * rdeg + b1_ref[...], 0.0)
    h_ref[...] = h
    s2_ref[...] = jnp.dot(h, w2s_ref[...], preferred_element_type=jnp.float32)
    rdeg_ref[...] = rdeg


def _stage_c(s1, aggp, d0, d1, b1, w2_self):
    return pl.pallas_call(
        _stage_c_kernel,
        grid=(GRID,),
        in_specs=[pl.BlockSpec((RB, DH), lambda i: (i, 0)),
                  pl.BlockSpec((NC, RB, DH), lambda i: (0, i, 0)),
                  pl.BlockSpec((RB, 1), lambda i: (i, 0)),
                  pl.BlockSpec((RB, 1), lambda i: (i, 0)),
                  pl.BlockSpec((1, DH), lambda i: (0, 0)),
                  pl.BlockSpec((DH, NCLS), lambda i: (0, 0))],
        out_specs=[pl.BlockSpec((RB, DH), lambda i: (i, 0)),
                   pl.BlockSpec((RB, NCLS), lambda i: (i, 0)),
                   pl.BlockSpec((RB, 1), lambda i: (i, 0))],
        out_shape=[jax.ShapeDtypeStruct((N, DH), jnp.float32),
                   jax.ShapeDtypeStruct((N, NCLS), jnp.float32),
                   jax.ShapeDtypeStruct((N, 1), jnp.float32)],
    )(s1, aggp, d0, d1, b1, w2_self)


def _stage_e_kernel(s2_ref, agg_ref, rdeg_ref, w2n_ref, b2_ref, o_ref):
    mean = (agg_ref[0] + agg_ref[1]) * rdeg_ref[...]
    t = s2_ref[...] + jnp.dot(mean, w2n_ref[...],
                              preferred_element_type=jnp.float32) + b2_ref[...]
    m = jnp.max(t, axis=1, keepdims=True)
    lse = m + jnp.log(jnp.sum(jnp.exp(t - m), axis=1, keepdims=True))
    o_ref[...] = t - lse


def _stage_e(s2, aggp, rdeg, w2_neigh, b2):
    return pl.pallas_call(
        _stage_e_kernel,
        grid=(GRID,),
        in_specs=[pl.BlockSpec((RB, NCLS), lambda i: (i, 0)),
                  pl.BlockSpec((NC, RB, DH), lambda i: (0, i, 0)),
                  pl.BlockSpec((RB, 1), lambda i: (i, 0)),
                  pl.BlockSpec((DH, NCLS), lambda i: (0, 0)),
                  pl.BlockSpec((1, NCLS), lambda i: (0, 0))],
        out_specs=pl.BlockSpec((RB, NCLS), lambda i: (i, 0)),
        out_shape=jax.ShapeDtypeStruct((N, NCLS), jnp.float32),
    )(s2, aggp, rdeg, w2_neigh, b2)


def kernel(x, edge_index, W1_self, W1_neigh, b1, W2_self, W2_neigh, b2):
    src = edge_index[0].astype(jnp.int32)
    dst = edge_index[1].astype(jnp.int32)
    padlen = EPAD - E
    spread = jnp.arange(padlen, dtype=jnp.int32) % (NPAD - N)
    src_p = jnp.concatenate([src, spread])
    dst_p = jnp.concatenate([dst, N + spread])

    s1, y1 = _stage_a(x, W1_self, W1_neigh)
    aggp, degp = _agg_deg(y1, src_p, dst_p)
    h, s2, rdeg = _stage_c(s1, aggp, degp[0, :N, None], degp[1, :N, None],
                           b1[None, :], W2_self)
    agg2 = _agg(h, src_p, dst_p)
    return _stage_e(s2, agg2, rdeg, W2_neigh, b2[None, :])


# speedup vs baseline: 22.6145x; 1.0232x over previous
import jax
import jax.numpy as jnp
from jax import lax
from jax.experimental import pallas as pl
from jax.experimental.pallas import tpu as pltpu
from jax.experimental.pallas import tpu_sc as plsc

N = 10000
E = 320000
DIN = 128
DH = 32
NCLS = 40

NC = 2
NS = 16
LANES = 16
NW = NC * NS
NPAD = 10240
EPW = 10240
EPAD = NW * EPW
CH = 1280
NCHUNK = EPW // CH
RPT = NPAD // NS


def _sc_mesh():
    return plsc.VectorSubcoreMesh(
        core_axis_name="c", subcore_axis_name="s",
        num_cores=NC, num_subcores=NS)


def _zero_rows(rows_v, nrows, width):
    z = jnp.zeros((LANES,), jnp.float32)
    def body(i, _):
        r4 = pl.multiple_of(i * 4, 4)
        for r in range(4):
            for j in range(width // LANES):
                rows_v[r4 + r, pl.ds(j * LANES, LANES)] = z
        return 0
    lax.fori_loop(0, nrows // 4, body, 0)


def _fill_1d(vec_v, n, value):
    v = jnp.full((LANES,), value, jnp.float32)
    def body(i, _):
        vec_v[pl.ds(pl.multiple_of(i * LANES, LANES), LANES)] = v
        return 0
    lax.fori_loop(0, n // LANES, body, 0)


def _edge_pipeline(y_hbm, src_hbm, dst_hbm, acc_sh, deg_sh,
                   idx_v, rows0, rows1, ones_v, zrow_v,
                   isem, gsem0, gsem1, s):
    c = lax.axis_index("c")
    wid = c * NS + s

    idx_copies = []
    for k in range(NCHUNK):
        base = wid * EPW + k * CH
        idx_copies.append(pltpu.async_copy(
            src_hbm.at[pl.ds(base, CH)], idx_v.at[0].at[k], isem))
        idx_copies.append(pltpu.async_copy(
            dst_hbm.at[pl.ds(base, CH)], idx_v.at[1].at[k], isem))

    _zero_rows(rows0, RPT, DH)
    strip = pl.ds(pl.multiple_of(s * RPT, RPT), RPT)
    pltpu.sync_copy(rows0.at[pl.ds(0, RPT)], acc_sh.at[strip])
    if deg_sh is not None:
        _fill_1d(ones_v, CH, 1.0)
        _fill_1d(zrow_v, RPT, 0.0)
        pltpu.sync_copy(zrow_v, deg_sh.at[strip])
    plsc.subcore_barrier()
    for cp in idx_copies:
        cp.wait()

    bufs = (rows0, rows1)
    gsems = (gsem0, gsem1)
    gathers = [None] * NCHUNK
    gathers[0] = pltpu.async_copy(y_hbm.at[idx_v.at[0].at[0]], bufs[0], gsem0)
    for k in range(NCHUNK):
        p = k % 2
        if k + 1 < NCHUNK:
            gathers[k + 1] = pltpu.async_copy(
                y_hbm.at[idx_v.at[0].at[k + 1]], bufs[1 - p], gsems[1 - p])
        gathers[k].wait()
        pltpu.sync_copy(bufs[p], acc_sh.at[idx_v.at[1].at[k]], add=True)
        if deg_sh is not None:
            pltpu.sync_copy(ones_v, deg_sh.at[idx_v.at[1].at[k]], add=True)
    plsc.subcore_barrier()
    return strip


def _agg_deg_body(y_hbm, src_hbm, dst_hbm, agg_out, deg_out,
                  idx_v, rows0, rows1, ones_v, zrow_v, acc_sh, deg_sh,
                  isem, gsem0, gsem1):
    c = lax.axis_index("c")
    s = lax.axis_index("s")
    strip = _edge_pipeline(y_hbm, src_hbm, dst_hbm, acc_sh, deg_sh,
                           idx_v, rows0, rows1, ones_v, zrow_v,
                           isem, gsem0, gsem1, s)
    pltpu.sync_copy(acc_sh.at[strip], agg_out.at[c, strip])
    pltpu.sync_copy(deg_sh.at[strip], deg_out.at[c, strip])


def _agg_body(y_hbm, src_hbm, dst_hbm, agg_out,
              idx_v, rows0, rows1, acc_sh, isem, gsem0, gsem1):
    c = lax.axis_index("c")
    s = lax.axis_index("s")
    strip = _edge_pipeline(y_hbm, src_hbm, dst_hbm, acc_sh, None,
                           idx_v, rows0, rows1, None, None,
                           isem, gsem0, gsem1, s)
    pltpu.sync_copy(acc_sh.at[strip], agg_out.at[c, strip])


def _agg_deg(y, src_p, dst_p):
    k = pl.kernel(
        _agg_deg_body,
        out_type=(jax.ShapeDtypeStruct((NC, NPAD, DH), jnp.float32),
                  jax.ShapeDtypeStruct((NC, NPAD), jnp.float32)),
        mesh=_sc_mesh(),
        compiler_params=pltpu.CompilerParams(use_tc_tiling_on_sc=False),
        scratch_types=[
            pltpu.VMEM((2, NCHUNK, CH), jnp.int32),
            pltpu.VMEM((CH, DH), jnp.float32),
            pltpu.VMEM((CH, DH), jnp.float32),
            pltpu.VMEM((CH,), jnp.float32),
            pltpu.VMEM((RPT,), jnp.float32),
            pltpu.VMEM_SHARED((NPAD, DH), jnp.float32),
            pltpu.VMEM_SHARED((NPAD,), jnp.float32),
            pltpu.SemaphoreType.DMA,
            pltpu.SemaphoreType.DMA,
            pltpu.SemaphoreType.DMA,
        ])
    return k(y, src_p, dst_p)


def _agg(y, src_p, dst_p):
    k = pl.kernel(
        _agg_body,
        out_type=jax.ShapeDtypeStruct((NC, NPAD, DH), jnp.float32),
        mesh=_sc_mesh(),
        compiler_params=pltpu.CompilerParams(use_tc_tiling_on_sc=False),
        scratch_types=[
            pltpu.VMEM((2, NCHUNK, CH), jnp.int32),
            pltpu.VMEM((CH, DH), jnp.float32),
            pltpu.VMEM((CH, DH), jnp.float32),
            pltpu.VMEM_SHARED((NPAD, DH), jnp.float32),
            pltpu.SemaphoreType.DMA,
            pltpu.SemaphoreType.DMA,
            pltpu.SemaphoreType.DMA,
        ])
    return k(y, src_p, dst_p)


RB = 5000
GRID = N // RB


def _stage_a_kernel(x_ref, ws_ref, wn_ref, s1_ref, y1_ref):
    xb = x_ref[...]
    s1_ref[...] = jnp.dot(xb, ws_ref[...], preferred_element_type=jnp.float32)
    y1_ref[...] = jnp.dot(xb, wn_ref[...], preferred_element_type=jnp.float32)


def _stage_a(x, w_self, w_neigh):
    return pl.pallas_call(
        _stage_a_kernel,
        grid=(GRID,),
        in_specs=[pl.BlockSpec((RB, DIN), lambda i: (i, 0)),
                  pl.BlockSpec((DIN, DH), lambda i: (0, 0)),
                  pl.BlockSpec((DIN, DH), lambda i: (0, 0))],
        out_specs=[pl.BlockSpec((RB, DH), lambda i: (i, 0)),
                   pl.BlockSpec((RB, DH), lambda i: (i, 0))],
        out_shape=[jax.ShapeDtypeStruct((N, DH), jnp.float32)] * 2,
    )(x, w_self, w_neigh)


def _stage_c_kernel(s1_ref, agg_ref, d0_ref, d1_ref, b1_ref, w2s_ref,
                    h_ref, s2_ref, rdeg_ref):
    deg = d0_ref[...] + d1_ref[...]
    rdeg = 1.0 / jnp.maximum(deg, 1.0)
    h = jnp.maximum(
        s1_ref[...] + (agg_ref[0] + agg_ref[1]) * rdeg + b1_ref[...], 0.0)
    h_ref[...] = h
    s2_ref[...] = jnp.dot(h, w2s_ref[...], preferred_element_type=jnp.float32)
    rdeg_ref[...] = rdeg


def _stage_c(s1, aggp, d0, d1, b1, w2_self):
    return pl.pallas_call(
        _stage_c_kernel,
        grid=(GRID,),
        in_specs=[pl.BlockSpec((RB, DH), lambda i: (i, 0)),
                  pl.BlockSpec((NC, RB, DH), lambda i: (0, i, 0)),
                  pl.BlockSpec((RB, 1), lambda i: (i, 0)),
                  pl.BlockSpec((RB, 1), lambda i: (i, 0)),
                  pl.BlockSpec((1, DH), lambda i: (0, 0)),
                  pl.BlockSpec((DH, NCLS), lambda i: (0, 0))],
        out_specs=[pl.BlockSpec((RB, DH), lambda i: (i, 0)),
                   pl.BlockSpec((RB, NCLS), lambda i: (i, 0)),
                   pl.BlockSpec((RB, 1), lambda i: (i, 0))],
        out_shape=[jax.ShapeDtypeStruct((N, DH), jnp.float32),
                   jax.ShapeDtypeStruct((N, NCLS), jnp.float32),
                   jax.ShapeDtypeStruct((N, 1), jnp.float32)],
    )(s1, aggp, d0, d1, b1, w2_self)


def _stage_e_kernel(s2_ref, agg_ref, rdeg_ref, w2n_ref, b2_ref, o_ref):
    mean = (agg_ref[0] + agg_ref[1]) * rdeg_ref[...]
    t = s2_ref[...] + jnp.dot(mean, w2n_ref[...],
                              preferred_element_type=jnp.float32) + b2_ref[...]
    m = jnp.max(t, axis=1, keepdims=True)
    lse = m + jnp.log(jnp.sum(jnp.exp(t - m), axis=1, keepdims=True))
    o_ref[...] = t - lse


def _stage_e(s2, aggp, rdeg, w2_neigh, b2):
    return pl.pallas_call(
        _stage_e_kernel,
        grid=(GRID,),
        in_specs=[pl.BlockSpec((RB, NCLS), lambda i: (i, 0)),
                  pl.BlockSpec((NC, RB, DH), lambda i: (0, i, 0)),
                  pl.BlockSpec((RB, 1), lambda i: (i, 0)),
                  pl.BlockSpec((DH, NCLS), lambda i: (0, 0)),
                  pl.BlockSpec((1, NCLS), lambda i: (0, 0))],
        out_specs=pl.BlockSpec((RB, NCLS), lambda i: (i, 0)),
        out_shape=jax.ShapeDtypeStruct((N, NCLS), jnp.float32),
    )(s2, aggp, rdeg, w2_neigh, b2)


def kernel(x, edge_index, W1_self, W1_neigh, b1, W2_self, W2_neigh, b2):
    src = edge_index[0].astype(jnp.int32)
    dst = edge_index[1].astype(jnp.int32)
    padlen = EPAD - E
    spread = jnp.arange(padlen, dtype=jnp.int32) % (NPAD - N)
    src_p = jnp.concatenate([src, spread])
    dst_p = jnp.concatenate([dst, N + spread])

    s1, y1 = _stage_a(x, W1_self, W1_neigh)
    aggp, degp = _agg_deg(y1, src_p, dst_p)
    h, s2, rdeg = _stage_c(s1, aggp, degp[0, :N, None], degp[1, :N, None],
                           b1[None, :], W2_self)
    agg2 = _agg(h, src_p, dst_p)
    return _stage_e(s2, agg2, rdeg, W2_neigh, b2[None, :])


# speedup vs baseline: 23.9403x; 1.0586x over previous
import jax
import jax.numpy as jnp
from jax import lax
from jax.experimental import pallas as pl
from jax.experimental.pallas import tpu as pltpu
from jax.experimental.pallas import tpu_sc as plsc

N = 10000
E = 320000
DIN = 128
DH = 32
NCLS = 40

NC = 2
NS = 16
LANES = 16
NW = NC * NS
NPAD = 10240
EPW = 10240
EPAD = NW * EPW
CH = 1280
NCHUNK = EPW // CH
RPT = NPAD // NS


def _sc_mesh():
    return plsc.VectorSubcoreMesh(
        core_axis_name="c", subcore_axis_name="s",
        num_cores=NC, num_subcores=NS)


def _zero_rows(rows_v, nrows, width):
    z = jnp.zeros((LANES,), jnp.float32)
    def body(i, _):
        r4 = pl.multiple_of(i * 4, 4)
        for r in range(4):
            for j in range(width // LANES):
                rows_v[r4 + r, pl.ds(j * LANES, LANES)] = z
        return 0
    lax.fori_loop(0, nrows // 4, body, 0)


def _fill_1d(vec_v, n, value):
    v = jnp.full((LANES,), value, jnp.float32)
    def body(i, _):
        vec_v[pl.ds(pl.multiple_of(i * LANES, LANES), LANES)] = v
        return 0
    lax.fori_loop(0, n // LANES, body, 0)


def _edge_pipeline(y_hbm, src_hbm, dst_hbm, acc_sh, deg_sh,
                   idx_v, rows0, rows1, ones_v, zrow_v,
                   isem, gsem0, gsem1, s):
    c = lax.axis_index("c")
    wid = c * NS + s

    idx_copies = []
    for k in range(NCHUNK):
        base = wid * EPW + k * CH
        idx_copies.append(pltpu.async_copy(
            src_hbm.at[pl.ds(base, CH)], idx_v.at[0].at[k], isem))
        idx_copies.append(pltpu.async_copy(
            dst_hbm.at[pl.ds(base, CH)], idx_v.at[1].at[k], isem))

    _zero_rows(rows0, RPT, DH)
    strip = pl.ds(pl.multiple_of(s * RPT, RPT), RPT)
    pltpu.sync_copy(rows0.at[pl.ds(0, RPT)], acc_sh.at[strip])
    if deg_sh is not None:
        _fill_1d(ones_v, CH, 1.0)
        _fill_1d(zrow_v, RPT, 0.0)
        pltpu.sync_copy(zrow_v, deg_sh.at[strip])
    plsc.subcore_barrier()
    for cp in idx_copies:
        cp.wait()

    bufs = (rows0, rows1)
    gsems = (gsem0, gsem1)
    gathers = [None] * NCHUNK
    gathers[0] = pltpu.async_copy(y_hbm.at[idx_v.at[0].at[0]], bufs[0], gsem0)
    for k in range(NCHUNK):
        p = k % 2
        if k + 1 < NCHUNK:
            gathers[k + 1] = pltpu.async_copy(
                y_hbm.at[idx_v.at[0].at[k + 1]], bufs[1 - p], gsems[1 - p])
        gathers[k].wait()
        pltpu.sync_copy(bufs[p], acc_sh.at[idx_v.at[1].at[k]], add=True)
        if deg_sh is not None:
            pltpu.sync_copy(ones_v, deg_sh.at[idx_v.at[1].at[k]], add=True)
    plsc.subcore_barrier()
    return strip


def _agg_deg_body(y_hbm, src_hbm, dst_hbm, agg_out, deg_out,
                  idx_v, rows0, rows1, ones_v, zrow_v, acc_sh, deg_sh,
                  isem, gsem0, gsem1):
    c = lax.axis_index("c")
    s = lax.axis_index("s")
    strip = _edge_pipeline(y_hbm, src_hbm, dst_hbm, acc_sh, deg_sh,
                           idx_v, rows0, rows1, ones_v, zrow_v,
                           isem, gsem0, gsem1, s)
    pltpu.sync_copy(acc_sh.at[strip], agg_out.at[c, strip])
    pltpu.sync_copy(deg_sh.at[strip], deg_out.at[c, strip])


def _agg_body(y_hbm, src_hbm, dst_hbm, agg_out,
              idx_v, rows0, rows1, acc_sh, isem, gsem0, gsem1):
    c = lax.axis_index("c")
    s = lax.axis_index("s")
    strip = _edge_pipeline(y_hbm, src_hbm, dst_hbm, acc_sh, None,
                           idx_v, rows0, rows1, None, None,
                           isem, gsem0, gsem1, s)
    pltpu.sync_copy(acc_sh.at[strip], agg_out.at[c, strip])


def _agg_deg(y, src_p, dst_p):
    k = pl.kernel(
        _agg_deg_body,
        out_type=(jax.ShapeDtypeStruct((NC, NPAD, DH), jnp.float32),
                  jax.ShapeDtypeStruct((NC, NPAD), jnp.float32)),
        mesh=_sc_mesh(),
        compiler_params=pltpu.CompilerParams(use_tc_tiling_on_sc=False),
        scratch_types=[
            pltpu.VMEM((2, NCHUNK, CH), jnp.int32),
            pltpu.VMEM((CH, DH), jnp.float32),
            pltpu.VMEM((CH, DH), jnp.float32),
            pltpu.VMEM((CH,), jnp.float32),
            pltpu.VMEM((RPT,), jnp.float32),
            pltpu.VMEM_SHARED((NPAD, DH), jnp.float32),
            pltpu.VMEM_SHARED((NPAD,), jnp.float32),
            pltpu.SemaphoreType.DMA,
            pltpu.SemaphoreType.DMA,
            pltpu.SemaphoreType.DMA,
        ])
    return k(y, src_p, dst_p)


def _agg(y, src_p, dst_p):
    k = pl.kernel(
        _agg_body,
        out_type=jax.ShapeDtypeStruct((NC, NPAD, DH), jnp.float32),
        mesh=_sc_mesh(),
        compiler_params=pltpu.CompilerParams(use_tc_tiling_on_sc=False),
        scratch_types=[
            pltpu.VMEM((2, NCHUNK, CH), jnp.int32),
            pltpu.VMEM((CH, DH), jnp.float32),
            pltpu.VMEM((CH, DH), jnp.float32),
            pltpu.VMEM_SHARED((NPAD, DH), jnp.float32),
            pltpu.SemaphoreType.DMA,
            pltpu.SemaphoreType.DMA,
            pltpu.SemaphoreType.DMA,
        ])
    return k(y, src_p, dst_p)


RB = 5000
GRID = N // RB


def _stage_a_kernel(x_ref, ws_ref, wn_ref, s1_ref, y1_ref):
    xb = x_ref[...]
    s1_ref[...] = jnp.dot(xb, ws_ref[...], preferred_element_type=jnp.float32)
    y1_ref[...] = jnp.dot(xb, wn_ref[...], preferred_element_type=jnp.float32)


def _stage_a(x, w_self, w_neigh):
    return pl.pallas_call(
        _stage_a_kernel,
        grid=(GRID,),
        in_specs=[pl.BlockSpec((RB, DIN), lambda i: (i, 0)),
                  pl.BlockSpec((DIN, DH), lambda i: (0, 0)),
                  pl.BlockSpec((DIN, DH), lambda i: (0, 0))],
        out_specs=[pl.BlockSpec((RB, DH), lambda i: (i, 0)),
                   pl.BlockSpec((RB, DH), lambda i: (i, 0))],
        out_shape=[jax.ShapeDtypeStruct((N, DH), jnp.float32)] * 2,
    )(x, w_self, w_neigh)


def _stage_c_kernel(s1_ref, agg_ref, d0_ref, d1_ref, b1_ref, w2s_ref,
                    h_ref, s2_ref, rdeg_ref):
    deg = d0_ref[...] + d1_ref[...]
    rdeg = 1.0 / jnp.maximum(deg, 1.0)
    h = jnp.maximum(
        s1_ref[...] + (agg_ref[0] + agg_ref[1]) * rdeg + b1_ref[...], 0.0)
    h_ref[...] = h
    s2_ref[...] = jnp.dot(h, w2s_ref[...], preferred_element_type=jnp.float32)
    rdeg_ref[...] = rdeg


def _stage_c(s1, aggp, d0, d1, b1, w2_self):
    return pl.pallas_call(
        _stage_c_kernel,
        grid=(GRID,),
        in_specs=[pl.BlockSpec((RB, DH), lambda i: (i, 0)),
                  pl.BlockSpec((NC, RB, DH), lambda i: (0, i, 0)),
                  pl.BlockSpec((RB, 1), lambda i: (i, 0)),
                  pl.BlockSpec((RB, 1), lambda i: (i, 0)),
                  pl.BlockSpec((1, DH), lambda i: (0, 0)),
                  pl.BlockSpec((DH, NCLS), lambda i: (0, 0))],
        out_specs=[pl.BlockSpec((RB, DH), lambda i: (i, 0)),
                   pl.BlockSpec((RB, NCLS), lambda i: (i, 0)),
                   pl.BlockSpec((RB, 1), lambda i: (i, 0))],
        out_shape=[jax.ShapeDtypeStruct((N, DH), jnp.float32),
                   jax.ShapeDtypeStruct((N, NCLS), jnp.float32),
                   jax.ShapeDtypeStruct((N, 1), jnp.float32)],
    )(s1, aggp, d0, d1, b1, w2_self)


def _stage_e_kernel(s2_ref, agg_ref, rdeg_ref, w2n_ref, b2_ref, o_ref):
    mean = (agg_ref[0] + agg_ref[1]) * rdeg_ref[...]
    t = s2_ref[...] + jnp.dot(mean, w2n_ref[...],
                              preferred_element_type=jnp.float32) + b2_ref[...]
    m = jnp.max(t, axis=1, keepdims=True)
    lse = m + jnp.log(jnp.sum(jnp.exp(t - m), axis=1, keepdims=True))
    o_ref[...] = t - lse


def _stage_e(s2, aggp, rdeg, w2_neigh, b2):
    return pl.pallas_call(
        _stage_e_kernel,
        grid=(GRID,),
        in_specs=[pl.BlockSpec((RB, NCLS), lambda i: (i, 0)),
                  pl.BlockSpec((NC, RB, DH), lambda i: (0, i, 0)),
                  pl.BlockSpec((RB, 1), lambda i: (i, 0)),
                  pl.BlockSpec((DH, NCLS), lambda i: (0, 0)),
                  pl.BlockSpec((1, NCLS), lambda i: (0, 0))],
        out_specs=pl.BlockSpec((RB, NCLS), lambda i: (i, 0)),
        out_shape=jax.ShapeDtypeStruct((N, NCLS), jnp.float32),
    )(s2, aggp, rdeg, w2_neigh, b2)


EB = E


def _split_kernel(e_ref, s_ref, d_ref):
    e = e_ref[...]
    s_ref[...] = e[0]
    d_ref[...] = e[1]


def _split_edges(edge_index):
    return pl.pallas_call(
        _split_kernel,
        grid=(E // EB,),
        in_specs=[pl.BlockSpec((2, EB), lambda i: (0, i))],
        out_specs=[pl.BlockSpec((EB,), lambda i: (i,)),
                   pl.BlockSpec((EB,), lambda i: (i,))],
        out_shape=[jax.ShapeDtypeStruct((E,), jnp.int32)] * 2,
    )(edge_index)


def kernel(x, edge_index, W1_self, W1_neigh, b1, W2_self, W2_neigh, b2):
    src, dst = _split_edges(edge_index)
    padlen = EPAD - E
    spread = jnp.arange(padlen, dtype=jnp.int32) % (NPAD - N)
    src_p = jnp.concatenate([src, spread])
    dst_p = jnp.concatenate([dst, N + spread])

    s1, y1 = _stage_a(x, W1_self, W1_neigh)
    aggp, degp = _agg_deg(y1, src_p, dst_p)
    h, s2, rdeg = _stage_c(s1, aggp, degp[0, :N, None], degp[1, :N, None],
                           b1[None, :], W2_self)
    agg2 = _agg(h, src_p, dst_p)
    return _stage_e(s2, agg2, rdeg, W2_neigh, b2[None, :])


# speedup vs baseline: 24.4054x; 1.0194x over previous
import jax
import jax.numpy as jnp
from jax import lax
from jax.experimental import pallas as pl
from jax.experimental.pallas import tpu as pltpu
from jax.experimental.pallas import tpu_sc as plsc

N = 10000
E = 320000
DIN = 128
DH = 32
NCLS = 40

NC = 2
NS = 16
LANES = 16
NW = NC * NS
NPAD = 10240
EPW = 10240
EPAD = NW * EPW
CH = 1280
NCHUNK = EPW // CH
RPT = NPAD // NS


def _sc_mesh():
    return plsc.VectorSubcoreMesh(
        core_axis_name="c", subcore_axis_name="s",
        num_cores=NC, num_subcores=NS)


def _zero_rows(rows_v, nrows, width):
    z = jnp.zeros((LANES,), jnp.float32)
    def body(i, _):
        r4 = pl.multiple_of(i * 4, 4)
        for r in range(4):
            for j in range(width // LANES):
                rows_v[r4 + r, pl.ds(j * LANES, LANES)] = z
        return 0
    lax.fori_loop(0, nrows // 4, body, 0)


def _fill_1d(vec_v, n, value):
    v = jnp.full((LANES,), value, jnp.float32)
    def body(i, _):
        vec_v[pl.ds(pl.multiple_of(i * LANES, LANES), LANES)] = v
        return 0
    lax.fori_loop(0, n // LANES, body, 0)


def _edge_pipeline(y_hbm, src_hbm, dst_hbm, acc_sh, deg_sh,
                   idx_v, rows0, rows1, ones_v, zrow_v,
                   isem, gsem0, gsem1, s):
    c = lax.axis_index("c")
    wid = c * NS + s

    idx_copies = []
    for k in range(NCHUNK):
        base = wid * EPW + k * CH
        idx_copies.append(pltpu.async_copy(
            src_hbm.at[pl.ds(base, CH)], idx_v.at[0].at[k], isem))
        idx_copies.append(pltpu.async_copy(
            dst_hbm.at[pl.ds(base, CH)], idx_v.at[1].at[k], isem))

    _zero_rows(rows0, RPT, DH)
    strip = pl.ds(pl.multiple_of(s * RPT, RPT), RPT)
    pltpu.sync_copy(rows0.at[pl.ds(0, RPT)], acc_sh.at[strip])
    if deg_sh is not None:
        _fill_1d(ones_v, CH, 1.0)
        _fill_1d(zrow_v, RPT, 0.0)
        pltpu.sync_copy(zrow_v, deg_sh.at[strip])
    plsc.subcore_barrier()
    for cp in idx_copies:
        cp.wait()

    bufs = (rows0, rows1)
    gsems = (gsem0, gsem1)
    gathers = [None] * NCHUNK
    gathers[0] = pltpu.async_copy(y_hbm.at[idx_v.at[0].at[0]], bufs[0], gsem0)
    for k in range(NCHUNK):
        p = k % 2
        if k + 1 < NCHUNK:
            gathers[k + 1] = pltpu.async_copy(
                y_hbm.at[idx_v.at[0].at[k + 1]], bufs[1 - p], gsems[1 - p])
        gathers[k].wait()
        pltpu.sync_copy(bufs[p], acc_sh.at[idx_v.at[1].at[k]], add=True)
        if deg_sh is not None:
            pltpu.sync_copy(ones_v, deg_sh.at[idx_v.at[1].at[k]], add=True)
    plsc.subcore_barrier()
    return strip


def _agg_deg_body(y_hbm, src_hbm, dst_hbm, agg_out, deg_out,
                  idx_v, rows0, rows1, ones_v, zrow_v, acc_sh, deg_sh,
                  isem, gsem0, gsem1):
    c = lax.axis_index("c")
    s = lax.axis_index("s")
    strip = _edge_pipeline(y_hbm, src_hbm, dst_hbm, acc_sh, deg_sh,
                           idx_v, rows0, rows1, ones_v, zrow_v,
                           isem, gsem0, gsem1, s)
    pltpu.sync_copy(acc_sh.at[strip], agg_out.at[c, strip])
    pltpu.sync_copy(deg_sh.at[strip], deg_out.at[c, strip])


def _agg_body(y_hbm, src_hbm, dst_hbm, agg_out,
              idx_v, rows0, rows1, acc_sh, isem, gsem0, gsem1):
    c = lax.axis_index("c")
    s = lax.axis_index("s")
    strip = _edge_pipeline(y_hbm, src_hbm, dst_hbm, acc_sh, None,
                           idx_v, rows0, rows1, None, None,
                           isem, gsem0, gsem1, s)
    pltpu.sync_copy(acc_sh.at[strip], agg_out.at[c, strip])


def _agg_deg(y, src_p, dst_p):
    k = pl.kernel(
        _agg_deg_body,
        out_type=(jax.ShapeDtypeStruct((NC, NPAD, DH), jnp.float32),
                  jax.ShapeDtypeStruct((NC, NPAD), jnp.float32)),
        mesh=_sc_mesh(),
        compiler_params=pltpu.CompilerParams(use_tc_tiling_on_sc=False),
        scratch_types=[
            pltpu.VMEM((2, NCHUNK, CH), jnp.int32),
            pltpu.VMEM((CH, DH), jnp.float32),
            pltpu.VMEM((CH, DH), jnp.float32),
            pltpu.VMEM((CH,), jnp.float32),
            pltpu.VMEM((RPT,), jnp.float32),
            pltpu.VMEM_SHARED((NPAD, DH), jnp.float32),
            pltpu.VMEM_SHARED((NPAD,), jnp.float32),
            pltpu.SemaphoreType.DMA,
            pltpu.SemaphoreType.DMA,
            pltpu.SemaphoreType.DMA,
        ])
    return k(y, src_p, dst_p)


def _agg(y, src_p, dst_p):
    k = pl.kernel(
        _agg_body,
        out_type=jax.ShapeDtypeStruct((NC, NPAD, DH), jnp.float32),
        mesh=_sc_mesh(),
        compiler_params=pltpu.CompilerParams(use_tc_tiling_on_sc=False),
        scratch_types=[
            pltpu.VMEM((2, NCHUNK, CH), jnp.int32),
            pltpu.VMEM((CH, DH), jnp.float32),
            pltpu.VMEM((CH, DH), jnp.float32),
            pltpu.VMEM_SHARED((NPAD, DH), jnp.float32),
            pltpu.SemaphoreType.DMA,
            pltpu.SemaphoreType.DMA,
            pltpu.SemaphoreType.DMA,
        ])
    return k(y, src_p, dst_p)


RB = 5000
GRID = N // RB


def _stage_a_kernel(x_ref, ws_ref, wn_ref, s1_ref, y1_ref):
    xb = x_ref[...]
    s1_ref[...] = jnp.dot(xb, ws_ref[...], preferred_element_type=jnp.float32)
    y1_ref[...] = jnp.dot(xb, wn_ref[...], preferred_element_type=jnp.float32)


def _stage_a(x, w_self, w_neigh):
    return pl.pallas_call(
        _stage_a_kernel,
        grid=(GRID,),
        in_specs=[pl.BlockSpec((RB, DIN), lambda i: (i, 0)),
                  pl.BlockSpec((DIN, DH), lambda i: (0, 0)),
                  pl.BlockSpec((DIN, DH), lambda i: (0, 0))],
        out_specs=[pl.BlockSpec((RB, DH), lambda i: (i, 0)),
                   pl.BlockSpec((RB, DH), lambda i: (i, 0))],
        out_shape=[jax.ShapeDtypeStruct((N, DH), jnp.float32)] * 2,
    )(x, w_self, w_neigh)


def _stage_c_kernel(s1_ref, agg_ref, d0_ref, d1_ref, b1_ref, w2s_ref,
                    h_ref, s2_ref, rdeg_ref):
    deg = d0_ref[...] + d1_ref[...]
    rdeg = 1.0 / jnp.maximum(deg, 1.0)
    h = jnp.maximum(
        s1_ref[...] + (agg_ref[0] + agg_ref[1]) * rdeg + b1_ref[...], 0.0)
    h_ref[...] = h
    s2_ref[...] = jnp.dot(h, w2s_ref[...], preferred_element_type=jnp.float32)
    rdeg_ref[...] = rdeg


def _stage_c(s1, aggp, d0, d1, b1, w2_self):
    return pl.pallas_call(
        _stage_c_kernel,
        grid=(GRID,),
        in_specs=[pl.BlockSpec((RB, DH), lambda i: (i, 0)),
                  pl.BlockSpec((NC, RB, DH), lambda i: (0, i, 0)),
                  pl.BlockSpec((RB, 1), lambda i: (i, 0)),
                  pl.BlockSpec((RB, 1), lambda i: (i, 0)),
                  pl.BlockSpec((1, DH), lambda i: (0, 0)),
                  pl.BlockSpec((DH, NCLS), lambda i: (0, 0))],
        out_specs=[pl.BlockSpec((RB, DH), lambda i: (i, 0)),
                   pl.BlockSpec((RB, NCLS), lambda i: (i, 0)),
                   pl.BlockSpec((RB, 1), lambda i: (i, 0))],
        out_shape=[jax.ShapeDtypeStruct((N, DH), jnp.float32),
                   jax.ShapeDtypeStruct((N, NCLS), jnp.float32),
                   jax.ShapeDtypeStruct((N, 1), jnp.float32)],
    )(s1, aggp, d0, d1, b1, w2_self)


def _stage_e_kernel(s2_ref, agg_ref, rdeg_ref, w2n_ref, b2_ref, o_ref):
    mean = (agg_ref[0] + agg_ref[1]) * rdeg_ref[...]
    t = s2_ref[...] + jnp.dot(mean, w2n_ref[...],
                              preferred_element_type=jnp.float32) + b2_ref[...]
    m = jnp.max(t, axis=1, keepdims=True)
    lse = m + jnp.log(jnp.sum(jnp.exp(t - m), axis=1, keepdims=True))
    o_ref[...] = t - lse


def _stage_e(s2, aggp, rdeg, w2_neigh, b2):
    return pl.pallas_call(
        _stage_e_kernel,
        grid=(GRID,),
        in_specs=[pl.BlockSpec((RB, NCLS), lambda i: (i, 0)),
                  pl.BlockSpec((NC, RB, DH), lambda i: (0, i, 0)),
                  pl.BlockSpec((RB, 1), lambda i: (i, 0)),
                  pl.BlockSpec((DH, NCLS), lambda i: (0, 0)),
                  pl.BlockSpec((1, NCLS), lambda i: (0, 0))],
        out_specs=pl.BlockSpec((RB, NCLS), lambda i: (i, 0)),
        out_shape=jax.ShapeDtypeStruct((N, NCLS), jnp.float32),
    )(s2, aggp, rdeg, w2_neigh, b2)


EB = E


def _split_kernel(e_ref, s_ref, d_ref):
    e = e_ref[...]
    s_ref[pl.ds(0, E)] = e[0]
    d_ref[pl.ds(0, E)] = e[1]
    spread = lax.iota(jnp.int32, EPAD - E) % (NPAD - N)
    s_ref[pl.ds(E, EPAD - E)] = spread
    d_ref[pl.ds(E, EPAD - E)] = N + spread


def _split_edges(edge_index):
    return pl.pallas_call(
        _split_kernel,
        grid=(1,),
        in_specs=[pl.BlockSpec((2, E), lambda i: (0, 0))],
        out_specs=[pl.BlockSpec((EPAD,), lambda i: (0,)),
                   pl.BlockSpec((EPAD,), lambda i: (0,))],
        out_shape=[jax.ShapeDtypeStruct((EPAD,), jnp.int32)] * 2,
    )(edge_index)


def kernel(x, edge_index, W1_self, W1_neigh, b1, W2_self, W2_neigh, b2):
    src_p, dst_p = _split_edges(edge_index)

    s1, y1 = _stage_a(x, W1_self, W1_neigh)
    aggp, degp = _agg_deg(y1, src_p, dst_p)
    h, s2, rdeg = _stage_c(s1, aggp, degp[0, :N, None], degp[1, :N, None],
                           b1[None, :], W2_self)
    agg2 = _agg(h, src_p, dst_p)
    return _stage_e(s2, agg2, rdeg, W2_neigh, b2[None, :])
